# baseline (speedup 1.0000x reference)
import functools

import jax
import jax.numpy as jnp
from jax import lax
from jax.experimental import pallas as pl
from jax.experimental.pallas import tpu as pltpu
from jax.experimental.pallas import tpu_sc as plsc

MAX_LEN = 8192
EMBED_DIM = 1024
SEQ_LEN = 4096

_NUM_WORKERS = 32
_ROWS_PER_WORKER = SEQ_LEN // _NUM_WORKERS

_CHUNKS = (16,) * 8
assert sum(_CHUNKS) == _ROWS_PER_WORKER
_MAXC = max(_CHUNKS)
_NCH = len(_CHUNKS)
_NBUF = 6
_OFFS = tuple(sum(_CHUNKS[:i]) for i in range(_NCH))

_mesh = plsc.VectorSubcoreMesh(core_axis_name="c", subcore_axis_name="s")


@functools.partial(
    pl.kernel,
    mesh=_mesh,
    out_type=jax.ShapeDtypeStruct((SEQ_LEN, EMBED_DIM), jnp.float32),
    scratch_types=(
        [pltpu.VMEM((_MAXC, EMBED_DIM), jnp.float32)] * _NBUF
        + [pltpu.SemaphoreType.DMA] * (2 * _NBUF)
    ),
)
def _copy_rows(table_hbm, out_hbm, *scratch):
    bufs = scratch[:_NBUF]
    sin = scratch[_NBUF : 2 * _NBUF]
    sout = scratch[2 * _NBUF :]
    wid = lax.axis_index("c") * 16 + lax.axis_index("s")
    base = wid * _ROWS_PER_WORKER

    def src(i):
        return table_hbm.at[pl.ds(base + _OFFS[i], _CHUNKS[i])]

    def dst(i):
        return out_hbm.at[pl.ds(base + _OFFS[i], _CHUNKS[i])]

    def buf(i):
        b = bufs[i % _NBUF]
        return b if _CHUNKS[i] == _MAXC else b.at[pl.ds(0, _CHUNKS[i])]

    in_c = [None] * _NCH
    out_c = [None] * _NCH
    for j in range(min(_NBUF, _NCH)):
        in_c[j] = pltpu.async_copy(src(j), buf(j), sin[j % _NBUF])
    for i in range(_NCH):
        b = i % _NBUF
        in_c[i].wait()
        out_c[i] = pltpu.async_copy(buf(i), dst(i), sout[b])
        j = i + _NBUF
        if j < _NCH:
            out_c[i].wait()
            in_c[j] = pltpu.async_copy(src(j), buf(j), sin[b])
    for i in range(max(0, _NCH - _NBUF), _NCH):
        out_c[i].wait()


def kernel(seq_len, table):
    del seq_len
    return _copy_rows(table)[None]
